# Initial kernel scaffold; baseline (speedup 1.0000x reference)
import jax
import jax.numpy as jnp
from jax.experimental import pallas as pl


def kernel(x_s, edge_index_s, edge_attr_s, batch_s, x_t, edge_index_t, edge_attr_t, batch_t, W_root, W_nbr, W_edge, b):
    raise NotImplementedError("write your pallas kernel here")


# speedup vs baseline: 108.7674x; 108.7674x over previous
import jax
import jax.numpy as jnp
from jax import lax
from jax.experimental import pallas as pl
from jax.experimental.pallas import tpu as pltpu
from jax.experimental.pallas import tpu_sc as plsc

_B = 5
_K = 10
_C = 128
_DE = 16
_N = 10000
_E = 160000

_NC = 2
_NS = 16
_NW = _NC * _NS

_EPT = _E // _NW
_CHUNK = 128
_EPT_PAD = ((_EPT + _CHUNK - 1) // _CHUNK) * _CHUNK
_NCHUNK = _EPT_PAD // _CHUNK
_TBL = 10240
_RZ = _TBL // _NS

_BIGI = 2 ** 30
_NEG = float('-inf')


def _sc_segsum_body(xw_hbm, src_hbm, dst_hbm, ew_hbm, zx_hbm,
                    agg_hbm, shx, rows_v, lin_v, si_v, di_v, sem):
    c = lax.axis_index("c")
    s = lax.axis_index("s")
    pltpu.sync_copy(zx_hbm, rows_v)
    for r in range(_RZ // _CHUNK):
        pltpu.sync_copy(rows_v, shx.at[pl.ds(s * _RZ + r * _CHUNK, _CHUNK)])
    plsc.subcore_barrier()
    base = (c * _NS + s) * _EPT_PAD

    def chunk(i, carry):
        off = base + i * _CHUNK
        pltpu.sync_copy(src_hbm.at[pl.ds(off, _CHUNK)], si_v)
        pltpu.sync_copy(dst_hbm.at[pl.ds(off, _CHUNK)], di_v)
        pltpu.async_copy(xw_hbm.at[si_v], rows_v, sem).wait()
        pltpu.sync_copy(rows_v, shx.at[di_v], add=True)
        pltpu.sync_copy(ew_hbm.at[pl.ds(off, _CHUNK)], lin_v)
        pltpu.sync_copy(lin_v, shx.at[di_v], add=True)
        return carry

    lax.fori_loop(0, _NCHUNK, chunk, 0)
    plsc.subcore_barrier()
    for r in range(_RZ // _CHUNK):
        off = s * _RZ + r * _CHUNK
        pltpu.sync_copy(shx.at[pl.ds(off, _CHUNK)], rows_v)
        pltpu.sync_copy(rows_v, agg_hbm.at[c, pl.ds(off, _CHUNK)])


@jax.jit
def _sc_segsum(xw, src_p, dst_p, ew_p, zx):
    mesh = plsc.VectorSubcoreMesh(core_axis_name="c", subcore_axis_name="s")
    return pl.kernel(
        _sc_segsum_body,
        out_type=jax.ShapeDtypeStruct((_NC, _TBL, _C), jnp.float32),
        mesh=mesh,
        scratch_types=[
            pltpu.VMEM_SHARED((_TBL, _C), jnp.float32),
            pltpu.VMEM((_CHUNK, _C), jnp.float32),
            pltpu.VMEM((_CHUNK, _C), jnp.float32),
            pltpu.VMEM((_CHUNK,), jnp.int32),
            pltpu.VMEM((_CHUNK,), jnp.int32),
            pltpu.SemaphoreType.DMA,
        ],
    )(xw, src_p, dst_p, ew_p, zx)


def _matmul_body(a_ref, w_ref, o_ref):
    o_ref[...] = jnp.dot(a_ref[...], w_ref[...],
                         preferred_element_type=jnp.float32)


def _matmul(a, w, br):
    m, ka = a.shape
    return pl.pallas_call(
        _matmul_body,
        grid=(m // br,),
        in_specs=[
            pl.BlockSpec((br, ka), lambda i: (i, 0)),
            pl.BlockSpec((ka, _C), lambda i: (0, 0)),
        ],
        out_specs=pl.BlockSpec((br, _C), lambda i: (i, 0)),
        out_shape=jax.ShapeDtypeStruct((m, _C), jnp.float32),
    )(a, w)


_BN = 1000


def _encode_body(x_ref, a2_ref, wr_ref, b_ref, h_ref):
    h_ref[...] = (jnp.dot(x_ref[...], wr_ref[...],
                          preferred_element_type=jnp.float32)
                  + a2_ref[0] + a2_ref[1] + b_ref[...])


@jax.jit
def _encode(x, agg, W_root, b2):
    return pl.pallas_call(
        _encode_body,
        grid=(_N // _BN,),
        in_specs=[
            pl.BlockSpec((_BN, _C), lambda i: (i, 0)),
            pl.BlockSpec((_NC, _BN, _C), lambda i: (0, i, 0)),
            pl.BlockSpec((_C, _C), lambda i: (0, 0)),
            pl.BlockSpec((1, _C), lambda i: (0, 0)),
        ],
        out_specs=pl.BlockSpec((_BN, _C), lambda i: (i, 0)),
        out_shape=jax.ShapeDtypeStruct((_N, _C), jnp.float32),
    )(x, agg, W_root, b2)


_BS = 400
_BT = 512
_NTP = ((_N + _BT - 1) // _BT) * _BT


def _match_body(ptr_s, ptr_t, hs_ref, ht_ref, s0_ref, si_ref):
    blk = pl.program_id(0)
    row0 = blk * _BS
    rows = row0 + lax.broadcasted_iota(jnp.int32, (_BS, 1), 0)
    lane16 = lax.broadcasted_iota(jnp.int32, (_BS, 16), 1)

    b_i = jnp.zeros((_BS, 1), jnp.int32)
    for bb in range(1, _B):
        b_i += (rows >= ptr_s[bb]).astype(jnp.int32)
    start = jnp.zeros((_BS, 1), jnp.int32)
    end = jnp.zeros((_BS, 1), jnp.int32)
    for bb in range(_B):
        sel = b_i == bb
        start = jnp.where(sel, ptr_t[bb], start)
        end = jnp.where(sel, ptr_t[bb + 1], end)
    count = end - start

    b_first = jnp.int32(0)
    b_last = jnp.int32(0)
    for bb in range(1, _B):
        b_first += (row0 >= ptr_s[bb]).astype(jnp.int32)
        b_last += (row0 + _BS - 1 >= ptr_s[bb]).astype(jnp.int32)
    lo = ptr_t[b_first]
    hi = ptr_t[b_last + 1]
    t0 = lo // _BT
    t1 = (hi + _BT - 1) // _BT

    hs = hs_ref[...]

    def tile_step(t, carry):
        rv, ri = carry
        htt = ht_ref[pl.ds(t * _BT, _BT), :]
        sims = lax.dot_general(hs, htt, (((1,), (1,)), ((), ())),
                               precision=lax.Precision.DEFAULT,
                               preferred_element_type=jnp.float32)
        gcol = t * _BT + lax.broadcasted_iota(jnp.int32, (_BS, _BT), 1)
        sims = jnp.where((gcol >= start) & (gcol < end), sims, _NEG)
        rptr = jnp.zeros((_BS, 1), jnp.int32)
        newv = []
        newi = []
        for _ in range(_K):
            tm = jnp.max(sims, axis=1, keepdims=True)
            ti = jnp.min(jnp.where(sims == tm, gcol, _BIGI), axis=1,
                         keepdims=True)
            rvk = jnp.max(jnp.where(lane16 == rptr, rv, _NEG), axis=1,
                          keepdims=True)
            rik = jnp.min(jnp.where(lane16 == rptr, ri, _BIGI), axis=1,
                          keepdims=True)
            take = (tm > rvk) | ((tm == rvk) & (ti < rik))
            newv.append(jnp.where(take, tm, rvk))
            newi.append(jnp.where(take, ti, rik))
            sims = jnp.where(take & (gcol == ti), _NEG, sims)
            rptr = rptr + jnp.where(take, 0, 1)
        pad_v = jnp.full((_BS, 16 - _K), _NEG, jnp.float32)
        pad_i = jnp.full((_BS, 16 - _K), _BIGI, jnp.int32)
        return (jnp.concatenate(newv + [pad_v], axis=1),
                jnp.concatenate(newi + [pad_i], axis=1))

    rv0 = jnp.full((_BS, 16), _NEG, jnp.float32)
    ri0 = jnp.full((_BS, 16), _BIGI, jnp.int32)
    rv, ri = lax.fori_loop(t0, t1, tile_step, (rv0, ri0))

    nonneg = (rv >= 0.0) & (lane16 < _K)
    p = jnp.sum(nonneg.astype(jnp.int32), axis=1, keepdims=True)
    z = jnp.minimum(_N - count, _K - p)
    sel = jnp.where(lane16 < p, lane16, lane16 - z)
    sel = jnp.clip(sel, 0, 15)
    gv = jnp.zeros((_BS, 16), jnp.float32)
    gi = jnp.zeros((_BS, 16), jnp.int32)
    for k in range(16):
        hit = sel == k
        gv = jnp.where(hit, rv[:, k:k + 1], gv)
        gi = jnp.where(hit, ri[:, k:k + 1], gi)
    zband = (lane16 >= p) & (lane16 < p + z)
    vfin = jnp.where(zband, 0.0, gv)
    ifin = jnp.where(zband, count + (lane16 - p), gi - start)
    valid = lane16 < _K
    vmax = jnp.max(jnp.where(valid, vfin, _NEG), axis=1, keepdims=True)
    ex = jnp.where(valid, jnp.exp(vfin - vmax), 0.0)
    s0 = ex / jnp.sum(ex, axis=1, keepdims=True)
    s0_ref[...] = jnp.where(valid, s0, 0.0)
    si_ref[...] = jnp.where(valid, ifin, 0)


@jax.jit
def _match(h_s, h_t, ptr_s8, ptr_t8):
    h_t = jnp.pad(h_t, ((0, _NTP - _N), (0, 0)))
    return pl.pallas_call(
        _match_body,
        grid=(_N // _BS,),
        in_specs=[
            pl.BlockSpec(memory_space=pltpu.SMEM),
            pl.BlockSpec(memory_space=pltpu.SMEM),
            pl.BlockSpec((_BS, _C), lambda i: (i, 0)),
            pl.BlockSpec((_NTP, _C), lambda i: (0, 0)),
        ],
        out_specs=[
            pl.BlockSpec((_BS, 16), lambda i: (i, 0)),
            pl.BlockSpec((_BS, 16), lambda i: (i, 0)),
        ],
        out_shape=[
            jax.ShapeDtypeStruct((_N, 16), jnp.float32),
            jax.ShapeDtypeStruct((_N, 16), jnp.int32),
        ],
    )(ptr_s8, ptr_t8, h_s, h_t)


def _pad_edges(edge_index, edge_attr):
    src = edge_index[0].reshape(_NW, _EPT)
    dst = edge_index[1].reshape(_NW, _EPT)
    ea = edge_attr.reshape(_NW, _EPT, _DE)
    padn = _EPT_PAD - _EPT
    src_p = jnp.concatenate(
        [src, jnp.zeros((_NW, padn), jnp.int32)], axis=1).reshape(-1)
    dst_p = jnp.concatenate(
        [dst, jnp.full((_NW, padn), _N, jnp.int32)], axis=1).reshape(-1)
    ea_p = jnp.concatenate(
        [ea, jnp.zeros((_NW, padn, _DE), jnp.float32)], axis=1).reshape(-1, _DE)
    return src_p, dst_p, ea_p


def _ptr(batch):
    ptr = jnp.searchsorted(batch, jnp.arange(_B + 1, dtype=jnp.int32),
                           side='left').astype(jnp.int32)
    return jnp.concatenate([ptr, jnp.full((2,), _N, jnp.int32)])


def kernel(x_s, edge_index_s, edge_attr_s, batch_s,
           x_t, edge_index_t, edge_attr_t, batch_t,
           W_root, W_nbr, W_edge, b):
    ptr_s8 = _ptr(batch_s)
    ptr_t8 = _ptr(batch_t)
    zx = jnp.zeros((_CHUNK, _C), jnp.float32)
    b2 = b.reshape(1, _C)

    src_s, dst_s, ea_s = _pad_edges(edge_index_s, edge_attr_s)
    src_t, dst_t, ea_t = _pad_edges(edge_index_t, edge_attr_t)

    xw_s = _matmul(x_s, W_nbr, _BN)
    xw_t = _matmul(x_t, W_nbr, _BN)
    ew_s = _matmul(ea_s, W_edge, 2048)
    ew_t = _matmul(ea_t, W_edge, 2048)

    agg_s = _sc_segsum(xw_s, src_s, dst_s, ew_s, zx)
    agg_t = _sc_segsum(xw_t, src_t, dst_t, ew_t, zx)

    h_s = _encode(x_s, agg_s, W_root, b2)
    h_t = _encode(x_t, agg_t, W_root, b2)

    s0p, sip = _match(h_s, h_t, ptr_s8, ptr_t8)
    return s0p[:, :_K], sip[:, :_K]
